# baseline (speedup 1.0000x reference)
import functools

import jax
import jax.numpy as jnp
from jax import lax
from jax.experimental import pallas as pl
from jax.experimental.pallas import tpu as pltpu
from jax.experimental.pallas import tpu_sc as plsc

D_NUM = 13
N_CAT = 26
D_TOK = 64
BATCH = 4096
N_TOK = D_NUM + N_CAT
LANES = 16
NW = 32
BW = BATCH // NW
G = 3
OCT = D_TOK // 8

CAT_GROUPS = [(c0, min(G, N_CAT - c0)) for c0 in range(0, N_CAT, G)]
NUM_GROUPS = [(j0, min(G, D_NUM - j0)) for j0 in range(0, D_NUM, G)]


def _make_kernel():
    mesh = plsc.VectorSubcoreMesh(core_axis_name="c", subcore_axis_name="s")

    @functools.partial(
        pl.kernel,
        out_type=jax.ShapeDtypeStruct((N_TOK * OCT, NW, 8, BW), jnp.float32),
        mesh=mesh,
        compiler_params=pltpu.CompilerParams(use_tc_tiling_on_sc=False,
                                             needs_layout_passes=False),
        scratch_types=[
            pltpu.VMEM((N_CAT, BW), jnp.int32),
            pltpu.VMEM((N_CAT, BW), jnp.int32),
            pltpu.VMEM((D_NUM, BW), jnp.float32),
            pltpu.VMEM((D_NUM * D_TOK,), jnp.float32),
            pltpu.VMEM((32,), jnp.int32),
            pltpu.VMEM((G, BW, D_TOK), jnp.float32),
            pltpu.VMEM((G, BW, D_TOK), jnp.float32),
            pltpu.VMEM((G * OCT, 8, BW), jnp.float32),
            pltpu.VMEM((G * OCT, 8, BW), jnp.float32),
            pltpu.SemaphoreType.DMA,
            pltpu.SemaphoreType.DMA,
            pltpu.SemaphoreType.DMA,
            pltpu.SemaphoreType.DMA,
        ],
    )
    def tok_kernel(xnT_hbm, xcT_hbm, wf_hbm, emb_hbm, off_hbm, u_hbm,
                   xc_v, idx_v, xn_v, wf_v, off_v,
                   cbufA, cbufB, tbufA, tbufB,
                   gsemA, gsemB, wsemA, wsemB):
        wid = lax.axis_index("s") * 2 + lax.axis_index("c")
        wb0 = wid * BW
        cbufs, gsems = (cbufA, cbufB), (gsemA, gsemB)
        tbufs, wsems = (tbufA, tbufB), (wsemA, wsemB)

        pltpu.sync_copy(xcT_hbm.at[:, pl.ds(wb0, BW)], xc_v)
        pltpu.sync_copy(xnT_hbm.at[:, pl.ds(wb0, BW)], xn_v)
        pltpu.sync_copy(wf_hbm, wf_v)
        pltpu.sync_copy(off_hbm, off_v)

        def idx_body(c, _):
            spl = plsc.load_gather(off_v, [jnp.full((LANES,), c, jnp.int32)])
            for k in range(BW // LANES):
                s = pl.ds(k * LANES, LANES)
                idx_v[c, s] = xc_v[c, s] + spl
            return 0
        lax.fori_loop(0, N_CAT, idx_body, 0)

        def fire(gi):
            c0, n = CAT_GROUPS[gi]
            cb, sem = cbufs[gi % 2], gsems[gi % 2]
            ds = []
            for cl in range(n):
                d = pltpu.make_async_copy(
                    emb_hbm.at[idx_v.at[c0 + cl]], cb.at[cl], sem)
                d.start()
                ds.append(d)
            return ds

        iv = [lax.iota(jnp.int32, LANES) + k * LANES
              for k in range(BW // LANES)]

        pend_w = [None, None]
        tb = 0

        def write_tiles(t0, n, tbuf, tb):
            d = pltpu.make_async_copy(
                tbuf.at[pl.ds(0, n * OCT)],
                u_hbm.at[pl.ds(t0 * OCT, n * OCT), wid],
                wsems[tb])
            d.start()
            return d

        gds = {0: fire(0), 1: fire(1)}

        for (j0, n) in NUM_GROUPS:
            tbuf = tbufs[tb]
            if pend_w[tb] is not None:
                pend_w[tb].wait()
            def num_body(r, _):
                tile0 = r // 8
                row = r % 8
                for jl in range(n):
                    widx = jnp.full((LANES,), (j0 + jl) * D_TOK + r,
                                    jnp.int32)
                    wspl = plsc.load_gather(wf_v, [widx])
                    for k in range(BW // LANES):
                        s = pl.ds(k * LANES, LANES)
                        tbuf[jl * OCT + tile0, row, s] = \
                            wspl * xn_v[j0 + jl, s]
                return 0
            lax.fori_loop(0, D_TOK, num_body, 0)
            pend_w[tb] = write_tiles(j0, n, tbuf, tb)
            tb ^= 1

        for gi, (c0, n) in enumerate(CAT_GROUPS):
            cb = cbufs[gi % 2]
            for d in gds.pop(gi):
                d.wait()
            tbuf = tbufs[tb]
            if pend_w[tb] is not None:
                pend_w[tb].wait()

            def tr_body(r, _):
                tile0 = r // 8
                row = r % 8
                dvec = jnp.full((LANES,), r, jnp.int32)
                for cl in range(n):
                    cvec = jnp.full((LANES,), cl, jnp.int32)
                    for k in range(BW // LANES):
                        v = plsc.load_gather(cb, [cvec, iv[k], dvec])
                        tbuf[cl * OCT + tile0, row,
                             pl.ds(k * LANES, LANES)] = v
                return 0
            lax.fori_loop(0, D_TOK, tr_body, 0)

            if gi + 2 < len(CAT_GROUPS):
                gds[gi + 2] = fire(gi + 2)
            pend_w[tb] = write_tiles(D_NUM + c0, n, tbuf, tb)
            tb ^= 1

        for d in pend_w:
            if d is not None:
                d.wait()

    return tok_kernel


def kernel(x_num, x_cat, weight, category_embeddings, category_offsets):
    xnT = x_num.T
    xcT = x_cat.astype(jnp.int32).T
    wf = weight.reshape(-1)
    offp = jnp.pad(category_offsets.astype(jnp.int32), (0, 32 - N_CAT))

    u = _make_kernel()(xnT, xcT, wf, category_embeddings, offp)
    v = jnp.transpose(u, (1, 3, 0, 2))
    return v.reshape(BATCH, N_TOK, D_TOK)


# speedup vs baseline: 1.6864x; 1.6864x over previous
import functools

import jax
import jax.numpy as jnp
from jax import lax
from jax.experimental import pallas as pl
from jax.experimental.pallas import tpu as pltpu
from jax.experimental.pallas import tpu_sc as plsc

D_NUM = 13
N_CAT = 26
D_TOK = 64
BATCH = 4096
N_TOK = D_NUM + N_CAT
LANES = 16
NW = 32
BW = BATCH // NW
G = 3
OCT = D_TOK // 8

CAT_GROUPS = [(c0, min(G, N_CAT - c0)) for c0 in range(0, N_CAT, G)]
NUM_GROUPS = [(j0, min(G, D_NUM - j0)) for j0 in range(0, D_NUM, G)]


def _make_kernel():
    mesh = plsc.VectorSubcoreMesh(core_axis_name="c", subcore_axis_name="s")

    @functools.partial(
        pl.kernel,
        out_type=jax.ShapeDtypeStruct((N_TOK * OCT, NW, 8, BW), jnp.float32),
        mesh=mesh,
        compiler_params=pltpu.CompilerParams(use_tc_tiling_on_sc=False,
                                             needs_layout_passes=False),
        scratch_types=[
            pltpu.VMEM((N_CAT, BW), jnp.int32),
            pltpu.VMEM((N_CAT, BW), jnp.int32),
            pltpu.VMEM((D_NUM, BW), jnp.float32),
            pltpu.VMEM((D_NUM * D_TOK,), jnp.float32),
            pltpu.VMEM((32,), jnp.int32),
            pltpu.VMEM((G, BW, D_TOK), jnp.float32),
            pltpu.VMEM((G, BW, D_TOK), jnp.float32),
            pltpu.VMEM((G * OCT, 8, BW), jnp.float32),
            pltpu.VMEM((G * OCT, 8, BW), jnp.float32),
            pltpu.SemaphoreType.DMA,
            pltpu.SemaphoreType.DMA,
            pltpu.SemaphoreType.DMA,
            pltpu.SemaphoreType.DMA,
        ],
    )
    def tok_kernel(xnT_hbm, xcT_hbm, wf_hbm, emb_hbm, off_hbm, u_hbm,
                   xc_v, idx_v, xn_v, wf_v, off_v,
                   cbufA, cbufB, tbufA, tbufB,
                   gsemA, gsemB, wsemA, wsemB):
        wid = lax.axis_index("s") * 2 + lax.axis_index("c")
        wb0 = wid * BW
        cbufs, gsems = (cbufA, cbufB), (gsemA, gsemB)
        tbufs, wsems = (tbufA, tbufB), (wsemA, wsemB)

        pltpu.sync_copy(xcT_hbm.at[:, pl.ds(wb0, BW)], xc_v)
        pltpu.sync_copy(xnT_hbm.at[:, pl.ds(wb0, BW)], xn_v)
        pltpu.sync_copy(wf_hbm, wf_v)
        pltpu.sync_copy(off_hbm, off_v)

        def idx_body(c, _):
            spl = plsc.load_gather(off_v, [jnp.full((LANES,), c, jnp.int32)])
            for k in range(BW // LANES):
                s = pl.ds(k * LANES, LANES)
                idx_v[c, s] = xc_v[c, s] + spl
            return 0
        lax.fori_loop(0, N_CAT, idx_body, 0)

        def fire(gi):
            c0, n = CAT_GROUPS[gi]
            cb, sem = cbufs[gi % 2], gsems[gi % 2]
            ds = []
            for cl in range(n):
                d = pltpu.make_async_copy(
                    emb_hbm.at[idx_v.at[c0 + cl]], cb.at[cl], sem)
                d.start()
                ds.append(d)
            return ds

        iv = [lax.iota(jnp.int32, LANES) + k * LANES
              for k in range(BW // LANES)]

        pend_w = [None, None]
        tb = 0

        def write_tiles(t0, n, tbuf, tb):
            d = pltpu.make_async_copy(
                tbuf.at[pl.ds(0, n * OCT)],
                u_hbm.at[pl.ds(t0 * OCT, n * OCT), wid],
                wsems[tb])
            d.start()
            return d

        gds = {0: fire(0), 1: fire(1)}

        for (j0, n) in NUM_GROUPS:
            tbuf = tbufs[tb]
            if pend_w[tb] is not None:
                pend_w[tb].wait()
            @plsc.parallel_loop(0, D_TOK, 1, unroll=4)
            def num_body(r):
                tile0 = r // 8
                row = r % 8
                for jl in range(n):
                    widx = jnp.full((LANES,), (j0 + jl) * D_TOK + r,
                                    jnp.int32)
                    wspl = plsc.load_gather(wf_v, [widx])
                    for k in range(BW // LANES):
                        s = pl.ds(k * LANES, LANES)
                        tbuf[jl * OCT + tile0, row, s] = \
                            wspl * xn_v[j0 + jl, s]
            pend_w[tb] = write_tiles(j0, n, tbuf, tb)
            tb ^= 1

        for gi, (c0, n) in enumerate(CAT_GROUPS):
            cb = cbufs[gi % 2]
            for d in gds.pop(gi):
                d.wait()
            tbuf = tbufs[tb]
            if pend_w[tb] is not None:
                pend_w[tb].wait()

            @plsc.parallel_loop(0, D_TOK, 1, unroll=4)
            def tr_body(r):
                tile0 = r // 8
                row = r % 8
                dvec = jnp.full((LANES,), r, jnp.int32)
                for cl in range(n):
                    cvec = jnp.full((LANES,), cl, jnp.int32)
                    for k in range(BW // LANES):
                        v = plsc.load_gather(cb, [cvec, iv[k], dvec])
                        tbuf[cl * OCT + tile0, row,
                             pl.ds(k * LANES, LANES)] = v

            if gi + 2 < len(CAT_GROUPS):
                gds[gi + 2] = fire(gi + 2)
            pend_w[tb] = write_tiles(D_NUM + c0, n, tbuf, tb)
            tb ^= 1

        for d in pend_w:
            if d is not None:
                d.wait()

    return tok_kernel


def kernel(x_num, x_cat, weight, category_embeddings, category_offsets):
    xnT = x_num.T
    xcT = x_cat.astype(jnp.int32).T
    wf = weight.reshape(-1)
    offp = jnp.pad(category_offsets.astype(jnp.int32), (0, 32 - N_CAT))

    u = _make_kernel()(xnT, xcT, wf, category_embeddings, offp)
    v = jnp.transpose(u, (1, 3, 0, 2))
    return v.reshape(BATCH, N_TOK, D_TOK)


# speedup vs baseline: 1.7023x; 1.0094x over previous
import functools

import jax
import jax.numpy as jnp
from jax import lax
from jax.experimental import pallas as pl
from jax.experimental.pallas import tpu as pltpu
from jax.experimental.pallas import tpu_sc as plsc

D_NUM = 13
N_CAT = 26
D_TOK = 64
BATCH = 4096
N_TOK = D_NUM + N_CAT
LANES = 16
NW = 32
BW = BATCH // NW
G = 3
OCT = D_TOK // 8

CAT_GROUPS = [(c0, min(G, N_CAT - c0)) for c0 in range(0, N_CAT, G)]
NUM_GROUPS = [(j0, min(G, D_NUM - j0)) for j0 in range(0, D_NUM, G)]


def _make_kernel():
    mesh = plsc.VectorSubcoreMesh(core_axis_name="c", subcore_axis_name="s")

    @functools.partial(
        pl.kernel,
        out_type=jax.ShapeDtypeStruct((N_TOK * OCT, NW, 8, BW), jnp.float32),
        mesh=mesh,
        compiler_params=pltpu.CompilerParams(use_tc_tiling_on_sc=False,
                                             needs_layout_passes=False),
        scratch_types=[
            pltpu.VMEM((N_CAT, BW), jnp.int32),
            pltpu.VMEM((N_CAT, BW), jnp.int32),
            pltpu.VMEM((D_NUM, BW), jnp.float32),
            pltpu.VMEM((D_NUM * D_TOK,), jnp.float32),
            pltpu.VMEM((32,), jnp.int32),
            pltpu.VMEM((G, BW, D_TOK), jnp.float32),
            pltpu.VMEM((G, BW, D_TOK), jnp.float32),
            pltpu.VMEM((G * OCT, 8, BW), jnp.float32),
            pltpu.VMEM((G * OCT, 8, BW), jnp.float32),
            pltpu.SemaphoreType.DMA,
            pltpu.SemaphoreType.DMA,
            pltpu.SemaphoreType.DMA,
            pltpu.SemaphoreType.DMA,
        ],
    )
    def tok_kernel(xnT_hbm, xcT_hbm, wf_hbm, emb_hbm, off_hbm, u_hbm,
                   xc_v, idx_v, xn_v, wf_v, off_v,
                   cbufA, cbufB, tbufA, tbufB,
                   gsemA, gsemB, wsemA, wsemB):
        wid = lax.axis_index("s") * 2 + lax.axis_index("c")
        wb0 = wid * BW
        cbufs, gsems = (cbufA, cbufB), (gsemA, gsemB)
        tbufs, wsems = (tbufA, tbufB), (wsemA, wsemB)

        pltpu.sync_copy(xcT_hbm.at[:, pl.ds(wb0, BW)], xc_v)
        pltpu.sync_copy(xnT_hbm.at[:, pl.ds(wb0, BW)], xn_v)
        pltpu.sync_copy(wf_hbm, wf_v)
        pltpu.sync_copy(off_hbm, off_v)

        def idx_body(c, _):
            spl = plsc.load_gather(off_v, [jnp.full((LANES,), c, jnp.int32)])
            for k in range(BW // LANES):
                s = pl.ds(k * LANES, LANES)
                idx_v[c, s] = xc_v[c, s] + spl
            return 0
        lax.fori_loop(0, N_CAT, idx_body, 0)

        def fire(gi):
            c0, n = CAT_GROUPS[gi]
            cb, sem = cbufs[gi % 2], gsems[gi % 2]
            ds = []
            for cl in range(n):
                d = pltpu.make_async_copy(
                    emb_hbm.at[idx_v.at[c0 + cl]], cb.at[cl], sem)
                d.start()
                ds.append(d)
            return ds

        iv = [lax.iota(jnp.int32, LANES) + k * LANES
              for k in range(BW // LANES)]

        pend_w = [None, None]
        tb = 0

        def write_tiles(t0, n, tbuf, tb):
            d = pltpu.make_async_copy(
                tbuf.at[pl.ds(0, n * OCT)],
                u_hbm.at[pl.ds(t0 * OCT, n * OCT), wid],
                wsems[tb])
            d.start()
            return d

        gds = {0: fire(0), 1: fire(1)}

        for (j0, n) in NUM_GROUPS:
            tbuf = tbufs[tb]
            if pend_w[tb] is not None:
                pend_w[tb].wait()
            xns = [[xn_v[j0 + jl, pl.ds(k * LANES, LANES)]
                    for k in range(BW // LANES)] for jl in range(n)]

            @plsc.parallel_loop(0, D_TOK, 1, unroll=4)
            def num_body(r):
                tile0 = r // 8
                row = r % 8
                for jl in range(n):
                    widx = jnp.full((LANES,), (j0 + jl) * D_TOK + r,
                                    jnp.int32)
                    wspl = plsc.load_gather(wf_v, [widx])
                    for k in range(BW // LANES):
                        s = pl.ds(k * LANES, LANES)
                        tbuf[jl * OCT + tile0, row, s] = wspl * xns[jl][k]
            pend_w[tb] = write_tiles(j0, n, tbuf, tb)
            tb ^= 1

        for gi, (c0, n) in enumerate(CAT_GROUPS):
            cb = cbufs[gi % 2]
            for d in gds.pop(gi):
                d.wait()
            tbuf = tbufs[tb]
            if pend_w[tb] is not None:
                pend_w[tb].wait()

            @plsc.parallel_loop(0, D_TOK, 1, unroll=4)
            def tr_body(r):
                tile0 = r // 8
                row = r % 8
                dvec = jnp.full((LANES,), r, jnp.int32)
                for cl in range(n):
                    cvec = jnp.full((LANES,), cl, jnp.int32)
                    for k in range(BW // LANES):
                        v = plsc.load_gather(cb, [cvec, iv[k], dvec])
                        tbuf[cl * OCT + tile0, row,
                             pl.ds(k * LANES, LANES)] = v

            if gi + 2 < len(CAT_GROUPS):
                gds[gi + 2] = fire(gi + 2)
            pend_w[tb] = write_tiles(D_NUM + c0, n, tbuf, tb)
            tb ^= 1

        for d in pend_w:
            if d is not None:
                d.wait()

    return tok_kernel


def kernel(x_num, x_cat, weight, category_embeddings, category_offsets):
    xnT = x_num.T
    xcT = x_cat.astype(jnp.int32).T
    wf = weight.reshape(-1)
    offp = jnp.pad(category_offsets.astype(jnp.int32), (0, 32 - N_CAT))

    u = _make_kernel()(xnT, xcT, wf, category_embeddings, offp)
    v = jnp.transpose(u, (1, 3, 0, 2))
    return v.reshape(BATCH, N_TOK, D_TOK)


# speedup vs baseline: 1.7061x; 1.0022x over previous
import functools

import jax
import jax.numpy as jnp
from jax import lax
from jax.experimental import pallas as pl
from jax.experimental.pallas import tpu as pltpu
from jax.experimental.pallas import tpu_sc as plsc

D_NUM = 13
N_CAT = 26
D_TOK = 64
BATCH = 4096
N_TOK = D_NUM + N_CAT
LANES = 16
NW = 32
BW = BATCH // NW
G = 3
OCT = D_TOK // 8

CAT_GROUPS = [(c0, min(G, N_CAT - c0)) for c0 in range(0, N_CAT, G)]
NUM_GROUPS = [(j0, min(G, D_NUM - j0)) for j0 in range(0, D_NUM, G)]


def _make_kernel():
    mesh = plsc.VectorSubcoreMesh(core_axis_name="c", subcore_axis_name="s")

    @functools.partial(
        pl.kernel,
        out_type=jax.ShapeDtypeStruct((N_TOK * OCT, NW, 8, BW), jnp.float32),
        mesh=mesh,
        compiler_params=pltpu.CompilerParams(use_tc_tiling_on_sc=False,
                                             needs_layout_passes=False),
        scratch_types=[
            pltpu.VMEM((N_CAT, BW), jnp.int32),
            pltpu.VMEM((N_CAT, BW), jnp.int32),
            pltpu.VMEM((D_NUM, BW), jnp.float32),
            pltpu.VMEM((D_NUM * D_TOK,), jnp.float32),
            pltpu.VMEM((32,), jnp.int32),
            pltpu.VMEM((G * BW, D_TOK), jnp.float32),
            pltpu.VMEM((G * BW, D_TOK), jnp.float32),
            pltpu.VMEM((G * OCT, 8, BW), jnp.float32),
            pltpu.VMEM((G * OCT, 8, BW), jnp.float32),
            pltpu.SemaphoreType.DMA,
            pltpu.SemaphoreType.DMA,
            pltpu.SemaphoreType.DMA,
            pltpu.SemaphoreType.DMA,
        ],
    )
    def tok_kernel(xnT_hbm, xcT_hbm, wf_hbm, emb_hbm, off_hbm, u_hbm,
                   xc_v, idx_v, xn_v, wf_v, off_v,
                   cbufA, cbufB, tbufA, tbufB,
                   gsemA, gsemB, wsemA, wsemB):
        wid = lax.axis_index("s") * 2 + lax.axis_index("c")
        wb0 = wid * BW
        cbufs, gsems = (cbufA, cbufB), (gsemA, gsemB)
        tbufs, wsems = (tbufA, tbufB), (wsemA, wsemB)

        pltpu.sync_copy(xcT_hbm.at[:, pl.ds(wb0, BW)], xc_v)
        pltpu.sync_copy(xnT_hbm.at[:, pl.ds(wb0, BW)], xn_v)
        pltpu.sync_copy(wf_hbm, wf_v)
        pltpu.sync_copy(off_hbm, off_v)

        def idx_body(c, _):
            spl = plsc.load_gather(off_v, [jnp.full((LANES,), c, jnp.int32)])
            for k in range(BW // LANES):
                s = pl.ds(k * LANES, LANES)
                idx_v[c, s] = xc_v[c, s] + spl
            return 0
        lax.fori_loop(0, N_CAT, idx_body, 0)

        def fire(gi):
            c0, n = CAT_GROUPS[gi]
            cb, sem = cbufs[gi % 2], gsems[gi % 2]
            ds = []
            for cl in range(n):
                d = pltpu.make_async_copy(
                    emb_hbm.at[idx_v.at[c0 + cl]],
                    cb.at[pl.ds(cl * BW, BW)], sem)
                d.start()
                ds.append(d)
            return ds

        iv = [lax.iota(jnp.int32, LANES) + k * LANES
              for k in range(BW // LANES)]

        pend_w = [None, None]
        tb = 0

        def write_tiles(t0, n, tbuf, tb):
            d = pltpu.make_async_copy(
                tbuf.at[pl.ds(0, n * OCT)],
                u_hbm.at[pl.ds(t0 * OCT, n * OCT), wid],
                wsems[tb])
            d.start()
            return d

        gds = {0: fire(0), 1: fire(1)}

        for (j0, n) in NUM_GROUPS:
            tbuf = tbufs[tb]
            if pend_w[tb] is not None:
                pend_w[tb].wait()
            xns = [[xn_v[j0 + jl, pl.ds(k * LANES, LANES)]
                    for k in range(BW // LANES)] for jl in range(n)]

            @plsc.parallel_loop(0, D_TOK, 1, unroll=4)
            def num_body(r):
                tile0 = r // 8
                row = r % 8
                for jl in range(n):
                    widx = jnp.full((LANES,), (j0 + jl) * D_TOK + r,
                                    jnp.int32)
                    wspl = plsc.load_gather(wf_v, [widx])
                    for k in range(BW // LANES):
                        s = pl.ds(k * LANES, LANES)
                        tbuf[jl * OCT + tile0, row, s] = wspl * xns[jl][k]
            pend_w[tb] = write_tiles(j0, n, tbuf, tb)
            tb ^= 1

        for gi, (c0, n) in enumerate(CAT_GROUPS):
            cb = cbufs[gi % 2]
            for d in gds.pop(gi):
                d.wait()
            tbuf = tbufs[tb]
            if pend_w[tb] is not None:
                pend_w[tb].wait()

            rowidx = [[iv[k] + cl * BW for k in range(BW // LANES)]
                      for cl in range(n)]

            @plsc.parallel_loop(0, D_TOK, 1, unroll=4)
            def tr_body(r):
                tile0 = r // 8
                row = r % 8
                dvec = jnp.full((LANES,), r, jnp.int32)
                for cl in range(n):
                    for k in range(BW // LANES):
                        v = plsc.load_gather(cb, [rowidx[cl][k], dvec])
                        tbuf[cl * OCT + tile0, row,
                             pl.ds(k * LANES, LANES)] = v

            if gi + 2 < len(CAT_GROUPS):
                gds[gi + 2] = fire(gi + 2)
            pend_w[tb] = write_tiles(D_NUM + c0, n, tbuf, tb)
            tb ^= 1

        for d in pend_w:
            if d is not None:
                d.wait()

    return tok_kernel


def kernel(x_num, x_cat, weight, category_embeddings, category_offsets):
    xnT = x_num.T
    xcT = x_cat.astype(jnp.int32).T
    wf = weight.reshape(-1)
    offp = jnp.pad(category_offsets.astype(jnp.int32), (0, 32 - N_CAT))

    u = _make_kernel()(xnT, xcT, wf, category_embeddings, offp)
    v = jnp.transpose(u, (1, 3, 0, 2))
    return v.reshape(BATCH, N_TOK, D_TOK)


# speedup vs baseline: 2.5793x; 1.5118x over previous
import functools

import jax
import jax.numpy as jnp
from jax import lax
from jax.experimental import pallas as pl
from jax.experimental.pallas import tpu as pltpu
from jax.experimental.pallas import tpu_sc as plsc

D_NUM = 13
N_CAT = 26
D_TOK = 64
BATCH = 4096
N_TOK = D_NUM + N_CAT
LANES = 16
NW = 32
BW = BATCH // NW
G = 3
OCT = D_TOK // 8
CPITCH = D_TOK + 8

CAT_GROUPS = [(c0, min(G, N_CAT - c0)) for c0 in range(0, N_CAT, G)]
NUM_GROUPS = [(j0, min(G, D_NUM - j0)) for j0 in range(0, D_NUM, G)]


def _make_kernel():
    mesh = plsc.VectorSubcoreMesh(core_axis_name="c", subcore_axis_name="s")

    @functools.partial(
        pl.kernel,
        out_type=jax.ShapeDtypeStruct((N_TOK * OCT, NW, 8, BW), jnp.float32),
        mesh=mesh,
        compiler_params=pltpu.CompilerParams(use_tc_tiling_on_sc=False,
                                             needs_layout_passes=False),
        scratch_types=[
            pltpu.VMEM((N_CAT, BW), jnp.int32),
            pltpu.VMEM((N_CAT, BW), jnp.int32),
            pltpu.VMEM((D_NUM, BW), jnp.float32),
            pltpu.VMEM((D_NUM * D_TOK,), jnp.float32),
            pltpu.VMEM((32,), jnp.int32),
            pltpu.VMEM((G * BW, CPITCH), jnp.float32),
            pltpu.VMEM((G * BW, CPITCH), jnp.float32),
            pltpu.VMEM((G * OCT, 8, BW), jnp.float32),
            pltpu.VMEM((G * OCT, 8, BW), jnp.float32),
            pltpu.SemaphoreType.DMA,
            pltpu.SemaphoreType.DMA,
            pltpu.SemaphoreType.DMA,
            pltpu.SemaphoreType.DMA,
        ],
    )
    def tok_kernel(xnT_hbm, xcT_hbm, wf_hbm, emb_hbm, off_hbm, u_hbm,
                   xc_v, idx_v, xn_v, wf_v, off_v,
                   cbufA, cbufB, tbufA, tbufB,
                   gsemA, gsemB, wsemA, wsemB):
        wid = lax.axis_index("s") * 2 + lax.axis_index("c")
        wb0 = wid * BW
        cbufs, gsems = (cbufA, cbufB), (gsemA, gsemB)
        tbufs, wsems = (tbufA, tbufB), (wsemA, wsemB)

        pltpu.sync_copy(xcT_hbm.at[:, pl.ds(wb0, BW)], xc_v)
        pltpu.sync_copy(xnT_hbm.at[:, pl.ds(wb0, BW)], xn_v)
        pltpu.sync_copy(wf_hbm, wf_v)
        pltpu.sync_copy(off_hbm, off_v)

        def idx_body(c, _):
            spl = plsc.load_gather(off_v, [jnp.full((LANES,), c, jnp.int32)])
            for k in range(BW // LANES):
                s = pl.ds(k * LANES, LANES)
                idx_v[c, s] = xc_v[c, s] + spl
            return 0
        lax.fori_loop(0, N_CAT, idx_body, 0)

        def fire(gi):
            c0, n = CAT_GROUPS[gi]
            cb, sem = cbufs[gi % 2], gsems[gi % 2]
            ds = []
            for cl in range(n):
                d = pltpu.make_async_copy(
                    emb_hbm.at[idx_v.at[c0 + cl]],
                    cb.at[pl.ds(cl * BW, BW)], sem)
                d.start()
                ds.append(d)
            return ds

        iv = [lax.iota(jnp.int32, LANES) + k * LANES
              for k in range(BW // LANES)]

        pend_w = [None, None]
        tb = 0

        def write_tiles(t0, n, tbuf, tb):
            d = pltpu.make_async_copy(
                tbuf.at[pl.ds(0, n * OCT)],
                u_hbm.at[pl.ds(t0 * OCT, n * OCT), wid],
                wsems[tb])
            d.start()
            return d

        gds = {0: fire(0), 1: fire(1)}

        for (j0, n) in NUM_GROUPS:
            tbuf = tbufs[tb]
            if pend_w[tb] is not None:
                pend_w[tb].wait()
            xns = [[xn_v[j0 + jl, pl.ds(k * LANES, LANES)]
                    for k in range(BW // LANES)] for jl in range(n)]

            @plsc.parallel_loop(0, D_TOK, 1, unroll=4)
            def num_body(r):
                tile0 = r // 8
                row = r % 8
                for jl in range(n):
                    widx = jnp.full((LANES,), (j0 + jl) * D_TOK + r,
                                    jnp.int32)
                    wspl = plsc.load_gather(wf_v, [widx])
                    for k in range(BW // LANES):
                        s = pl.ds(k * LANES, LANES)
                        tbuf[jl * OCT + tile0, row, s] = wspl * xns[jl][k]
            pend_w[tb] = write_tiles(j0, n, tbuf, tb)
            tb ^= 1

        for gi, (c0, n) in enumerate(CAT_GROUPS):
            cb = cbufs[gi % 2]
            for d in gds.pop(gi):
                d.wait()
            tbuf = tbufs[tb]
            if pend_w[tb] is not None:
                pend_w[tb].wait()

            rowidx = [[iv[k] + cl * BW for k in range(BW // LANES)]
                      for cl in range(n)]

            @plsc.parallel_loop(0, D_TOK, 1, unroll=4)
            def tr_body(r):
                tile0 = r // 8
                row = r % 8
                dvec = jnp.full((LANES,), r, jnp.int32)
                for cl in range(n):
                    for k in range(BW // LANES):
                        v = plsc.load_gather(cb, [rowidx[cl][k], dvec])
                        tbuf[cl * OCT + tile0, row,
                             pl.ds(k * LANES, LANES)] = v

            if gi + 2 < len(CAT_GROUPS):
                gds[gi + 2] = fire(gi + 2)
            pend_w[tb] = write_tiles(D_NUM + c0, n, tbuf, tb)
            tb ^= 1

        for d in pend_w:
            if d is not None:
                d.wait()

    return tok_kernel


def kernel(x_num, x_cat, weight, category_embeddings, category_offsets):
    xnT = x_num.T
    xcT = x_cat.astype(jnp.int32).T
    wf = weight.reshape(-1)
    offp = jnp.pad(category_offsets.astype(jnp.int32), (0, 32 - N_CAT))

    emb65 = jnp.pad(category_embeddings, ((0, 0), (0, CPITCH - D_TOK)))
    u = _make_kernel()(xnT, xcT, wf, emb65, offp)
    v = jnp.transpose(u, (1, 3, 0, 2))
    return v.reshape(BATCH, N_TOK, D_TOK)
